# initial kernel scaffold (unmeasured)
import jax
import jax.numpy as jnp
from jax import lax
from jax.experimental import pallas as pl
from jax.experimental.pallas import tpu as pltpu


def kernel(
    x,
):
    def body(*refs):
        pass

    out_shape = jax.ShapeDtypeStruct(..., jnp.float32)
    return pl.pallas_call(body, out_shape=out_shape)(...)


# baseline (device time: 149414 ns/iter reference)
import jax
import jax.numpy as jnp
from jax import lax
from jax.experimental import pallas as pl
from jax.experimental.pallas import tpu as pltpu

N_Z = 4


def kernel(x):
    m, n = x.shape

    def body(x_ref, out_ref, comm_ref, send_sems, recv_sems):
        my_x = lax.axis_index("x")
        my_y = lax.axis_index("y")
        my_z = lax.axis_index("z")
        left = (my_z - 1) % N_Z
        right = (my_z + 1) % N_Z

        barrier_sem = pltpu.get_barrier_semaphore()
        for nbr in [left, right]:
            pl.semaphore_signal(
                barrier_sem,
                inc=1,
                device_id=(my_x, my_y, nbr),
                device_id_type=pl.DeviceIdType.MESH,
            )
        pl.semaphore_wait(barrier_sem, 2)

        out_ref[...] = x_ref[...]

        for h in range(N_Z - 1):
            src = x_ref if h == 0 else comm_ref.at[h - 1]
            rdma = pltpu.make_async_remote_copy(
                src_ref=src,
                dst_ref=comm_ref.at[h],
                send_sem=send_sems.at[h],
                recv_sem=recv_sems.at[h],
                device_id=(my_x, my_y, right),
                device_id_type=pl.DeviceIdType.MESH,
            )
            rdma.start()
            rdma.wait()
            out_ref[...] += comm_ref[h]

    return pl.pallas_call(
        body,
        out_shape=jax.ShapeDtypeStruct((m, n), x.dtype),
        in_specs=[pl.BlockSpec(memory_space=pltpu.VMEM)],
        out_specs=pl.BlockSpec(memory_space=pltpu.VMEM),
        scratch_shapes=[
            pltpu.VMEM((N_Z - 1, m, n), x.dtype),
            pltpu.SemaphoreType.DMA((N_Z - 1,)),
            pltpu.SemaphoreType.DMA((N_Z - 1,)),
        ],
        compiler_params=pltpu.CompilerParams(collective_id=0),
    )(x)


# device time: 100215 ns/iter; 1.4909x vs baseline; 1.4909x over previous
import jax
import jax.numpy as jnp
from jax import lax
from jax.experimental import pallas as pl
from jax.experimental.pallas import tpu as pltpu

N_Z = 4
C = 8


def kernel(x):
    m, n = x.shape
    ch = m // C

    def body(x_ref, out_ref, rbuf, s1, r1, s2, r2, s3, r3):
        my_x = lax.axis_index("x")
        my_y = lax.axis_index("y")
        my_z = lax.axis_index("z")
        is_end = (my_z == 0) | (my_z == 3)
        mid_of_end = jnp.where(my_z == 0, 1, 2)
        end_of_mid = jnp.where(my_z == 1, 0, 3)
        other_mid = jnp.where(my_z == 1, 2, 1)

        def rows(c):
            return pl.ds(c * ch, ch)

        barrier_sem = pltpu.get_barrier_semaphore()

        @pl.when(is_end)
        def _():
            pl.semaphore_signal(
                barrier_sem, inc=1,
                device_id=(my_x, my_y, mid_of_end),
                device_id_type=pl.DeviceIdType.MESH,
            )
            pl.semaphore_wait(barrier_sem, 1)

        @pl.when(~is_end)
        def _():
            for nbr in (end_of_mid, other_mid):
                pl.semaphore_signal(
                    barrier_sem, inc=1,
                    device_id=(my_x, my_y, nbr),
                    device_id_type=pl.DeviceIdType.MESH,
                )
            pl.semaphore_wait(barrier_sem, 2)

        @pl.when(is_end)
        def _():
            sends = []
            for c in range(C):
                snd = pltpu.make_async_remote_copy(
                    src_ref=x_ref.at[rows(c), :],
                    dst_ref=out_ref.at[rows(c), :],
                    send_sem=s1.at[c],
                    recv_sem=r1.at[c],
                    device_id=(my_x, my_y, mid_of_end),
                    device_id_type=pl.DeviceIdType.MESH,
                )
                snd.start()
                sends.append(snd)
            for c in range(C):
                fin = pltpu.make_async_remote_copy(
                    src_ref=out_ref.at[rows(c), :],
                    dst_ref=out_ref.at[rows(c), :],
                    send_sem=s3.at[c],
                    recv_sem=r3.at[c],
                    device_id=(my_x, my_y, mid_of_end),
                    device_id_type=pl.DeviceIdType.MESH,
                )
                fin.wait_recv()
            for snd in sends:
                snd.wait_send()

        @pl.when(~is_end)
        def _():
            exchanges = []
            for c in range(C):
                rx = pltpu.make_async_remote_copy(
                    src_ref=x_ref.at[rows(c), :],
                    dst_ref=out_ref.at[rows(c), :],
                    send_sem=s1.at[c],
                    recv_sem=r1.at[c],
                    device_id=(my_x, my_y, end_of_mid),
                    device_id_type=pl.DeviceIdType.MESH,
                )
                rx.wait_recv()
                out_ref[rows(c), :] = out_ref[rows(c), :] + x_ref[rows(c), :]
                ex = pltpu.make_async_remote_copy(
                    src_ref=out_ref.at[rows(c), :],
                    dst_ref=rbuf.at[c],
                    send_sem=s2.at[c],
                    recv_sem=r2.at[c],
                    device_id=(my_x, my_y, other_mid),
                    device_id_type=pl.DeviceIdType.MESH,
                )
                ex.start()
                exchanges.append(ex)
            finals = []
            for c in range(C):
                exchanges[c].wait()
                out_ref[rows(c), :] = out_ref[rows(c), :] + rbuf[c]
                fin = pltpu.make_async_remote_copy(
                    src_ref=out_ref.at[rows(c), :],
                    dst_ref=out_ref.at[rows(c), :],
                    send_sem=s3.at[c],
                    recv_sem=r3.at[c],
                    device_id=(my_x, my_y, end_of_mid),
                    device_id_type=pl.DeviceIdType.MESH,
                )
                fin.start()
                finals.append(fin)
            for fin in finals:
                fin.wait_send()

    return pl.pallas_call(
        body,
        out_shape=jax.ShapeDtypeStruct((m, n), x.dtype),
        in_specs=[pl.BlockSpec(memory_space=pltpu.VMEM)],
        out_specs=pl.BlockSpec(memory_space=pltpu.VMEM),
        scratch_shapes=[
            pltpu.VMEM((C, ch, n), x.dtype),
            pltpu.SemaphoreType.DMA((C,)),
            pltpu.SemaphoreType.DMA((C,)),
            pltpu.SemaphoreType.DMA((C,)),
            pltpu.SemaphoreType.DMA((C,)),
            pltpu.SemaphoreType.DMA((C,)),
            pltpu.SemaphoreType.DMA((C,)),
        ],
        compiler_params=pltpu.CompilerParams(collective_id=0),
    )(x)


# device time: 45763 ns/iter; 3.2650x vs baseline; 2.1899x over previous
import jax
import jax.numpy as jnp
from jax import lax
from jax.experimental import pallas as pl
from jax.experimental.pallas import tpu as pltpu

N_Z = 4
C = 8


def kernel(x):
    m, n = x.shape
    half = m // 2
    ch = half // C

    def body(x_ref, out_ref, rbuf, s1, r1, s2, r2, s3, r3, ys, yr):
        my_x = lax.axis_index("x")
        my_y = lax.axis_index("y")
        my_z = lax.axis_index("z")
        is_end = (my_z == 0) | (my_z == 3)
        mid_of_end = jnp.where(my_z == 0, 1, 2)
        end_of_mid = jnp.where(my_z == 1, 0, 3)
        other_mid = jnp.where(my_z == 1, 2, 1)
        y_par = my_y + 1 - 2 * (my_y % 2)

        base = (my_y % 2) * half
        pbase = half - base

        def own(c):
            return pl.ds(base + c * ch, ch)

        def par(c):
            return pl.ds(pbase + c * ch, ch)

        barrier_sem = pltpu.get_barrier_semaphore()

        def sig(dev_id):
            pl.semaphore_signal(
                barrier_sem, inc=1, device_id=dev_id,
                device_id_type=pl.DeviceIdType.MESH,
            )

        @pl.when(is_end)
        def _():
            sig((my_x, my_y, mid_of_end))
            sig((my_x, y_par, my_z))
            pl.semaphore_wait(barrier_sem, 2)

        @pl.when(~is_end)
        def _():
            sig((my_x, my_y, end_of_mid))
            sig((my_x, my_y, other_mid))
            sig((my_x, y_par, my_z))
            pl.semaphore_wait(barrier_sem, 3)

        def y_send(c):
            snd = pltpu.make_async_remote_copy(
                src_ref=out_ref.at[own(c), :],
                dst_ref=out_ref.at[own(c), :],
                send_sem=ys.at[c],
                recv_sem=yr.at[c],
                device_id=(my_x, y_par, my_z),
                device_id_type=pl.DeviceIdType.MESH,
            )
            snd.start()
            return snd

        def y_recv_wait(c):
            rcv = pltpu.make_async_remote_copy(
                src_ref=out_ref.at[par(c), :],
                dst_ref=out_ref.at[par(c), :],
                send_sem=ys.at[c],
                recv_sem=yr.at[c],
                device_id=(my_x, y_par, my_z),
                device_id_type=pl.DeviceIdType.MESH,
            )
            rcv.wait_recv()

        @pl.when(is_end)
        def _():
            sends = []
            for c in range(C):
                snd = pltpu.make_async_remote_copy(
                    src_ref=x_ref.at[own(c), :],
                    dst_ref=out_ref.at[own(c), :],
                    send_sem=s1.at[c],
                    recv_sem=r1.at[c],
                    device_id=(my_x, my_y, mid_of_end),
                    device_id_type=pl.DeviceIdType.MESH,
                )
                snd.start()
                sends.append(snd)
            for c in range(C):
                fin = pltpu.make_async_remote_copy(
                    src_ref=out_ref.at[own(c), :],
                    dst_ref=out_ref.at[own(c), :],
                    send_sem=s3.at[c],
                    recv_sem=r3.at[c],
                    device_id=(my_x, my_y, mid_of_end),
                    device_id_type=pl.DeviceIdType.MESH,
                )
                fin.wait_recv()
                sends.append(y_send(c))
            for c in range(C):
                y_recv_wait(c)
            for snd in sends:
                snd.wait_send()

        @pl.when(~is_end)
        def _():
            exchanges = []
            pending = []

            def finish(j):
                exchanges[j].wait()
                out_ref[own(j), :] = out_ref[own(j), :] + rbuf[j]
                fin = pltpu.make_async_remote_copy(
                    src_ref=out_ref.at[own(j), :],
                    dst_ref=out_ref.at[own(j), :],
                    send_sem=s3.at[j],
                    recv_sem=r3.at[j],
                    device_id=(my_x, my_y, end_of_mid),
                    device_id_type=pl.DeviceIdType.MESH,
                )
                fin.start()
                pending.append(fin)
                pending.append(y_send(j))

            for c in range(C):
                rx = pltpu.make_async_remote_copy(
                    src_ref=x_ref.at[own(c), :],
                    dst_ref=out_ref.at[own(c), :],
                    send_sem=s1.at[c],
                    recv_sem=r1.at[c],
                    device_id=(my_x, my_y, end_of_mid),
                    device_id_type=pl.DeviceIdType.MESH,
                )
                rx.wait_recv()
                out_ref[own(c), :] = out_ref[own(c), :] + x_ref[own(c), :]
                ex = pltpu.make_async_remote_copy(
                    src_ref=out_ref.at[own(c), :],
                    dst_ref=rbuf.at[c],
                    send_sem=s2.at[c],
                    recv_sem=r2.at[c],
                    device_id=(my_x, my_y, other_mid),
                    device_id_type=pl.DeviceIdType.MESH,
                )
                ex.start()
                exchanges.append(ex)
                if c >= 1:
                    finish(c - 1)
            finish(C - 1)
            for c in range(C):
                y_recv_wait(c)
            for op in pending:
                op.wait_send()

    return pl.pallas_call(
        body,
        out_shape=jax.ShapeDtypeStruct((m, n), x.dtype),
        in_specs=[pl.BlockSpec(memory_space=pltpu.VMEM)],
        out_specs=pl.BlockSpec(memory_space=pltpu.VMEM),
        scratch_shapes=[
            pltpu.VMEM((C, ch, n), x.dtype),
            pltpu.SemaphoreType.DMA((C,)),
            pltpu.SemaphoreType.DMA((C,)),
            pltpu.SemaphoreType.DMA((C,)),
            pltpu.SemaphoreType.DMA((C,)),
            pltpu.SemaphoreType.DMA((C,)),
            pltpu.SemaphoreType.DMA((C,)),
            pltpu.SemaphoreType.DMA((C,)),
            pltpu.SemaphoreType.DMA((C,)),
        ],
        compiler_params=pltpu.CompilerParams(collective_id=0),
    )(x)
